# baseline (speedup 1.0000x reference)
import functools

import jax
import jax.numpy as jnp
from jax import lax
from jax.experimental import pallas as pl
from jax.experimental.pallas import tpu as pltpu
from jax.experimental.pallas import tpu_sc as plsc

NBINS_X = 1000
MAX_Y_TOKENS = 201
EMBED_DIM = 64
MAX_LEN = 200
B = 4096
T = 200
N = B * T

LANES = 16
BBLK = 128
NBH = B // BBLK
NTT = T // 8
OUT_ROWS = T * (EMBED_DIM // 8) * NBH


def _make_kernel():
    info = plsc.get_sparse_core_info()
    nc, ns = info.num_cores, info.num_subcores
    nw = nc * ns
    assert nw == NBH

    mesh = plsc.VectorSubcoreMesh(core_axis_name="c", subcore_axis_name="s")

    f32 = jnp.float32
    i32 = jnp.int32

    @functools.partial(
        pl.kernel,
        mesh=mesh,
        out_type=jax.ShapeDtypeStruct((OUT_ROWS, 1024), f32),
        compiler_params=pltpu.CompilerParams(use_tc_tiling_on_sc=False,
                                             needs_layout_passes=False),
        scratch_types=[
            pltpu.VMEM((NBINS_X, EMBED_DIM), f32),
            pltpu.VMEM((MAX_Y_TOKENS, EMBED_DIM), f32),
            pltpu.VMEM((MAX_LEN, EMBED_DIM), f32),
            pltpu.VMEM((BBLK,), i32), pltpu.VMEM((BBLK,), i32),
            pltpu.VMEM((BBLK,), i32), pltpu.VMEM((BBLK,), i32),
            pltpu.VMEM((8 * 1024,), f32), pltpu.VMEM((8 * 1024,), f32),
            pltpu.SemaphoreType.DMA, pltpu.SemaphoreType.DMA,
            pltpu.SemaphoreType.DMA, pltpu.SemaphoreType.DMA,
        ],
    )
    def k(x_tok, y_tok, x_table, y_table, pos_table, out,
          xv, yv, pv, xt_a, xt_b, yt_a, yt_b, acc_a, acc_b,
          st_a, st_b, so_a, so_b):
        w = lax.axis_index("s") * nc + lax.axis_index("c")

        xt = (xt_a, xt_b)
        yt = (yt_a, yt_b)
        acc = (acc_a, acc_b)
        st = (st_a, st_b)
        so = (so_a, so_b)

        pltpu.sync_copy(x_table, xv)
        pltpu.sync_copy(y_table, yv)
        pltpu.sync_copy(pos_table, pv)

        def tok_off(t):
            return ((t // 8) * NBH + w) * 1024 + (t % 8) * BBLK

        def start_tok(t, p):
            off = tok_off(t)
            pltpu.async_copy(x_tok.at[pl.ds(off, BBLK)], xt[p], st[p])
            pltpu.async_copy(y_tok.at[pl.ds(off, BBLK)], yt[p], st[p])

        def wait_tok(p):
            pltpu.make_async_copy(x_tok.at[pl.ds(0, BBLK)], xt[p], st[p]).wait()
            pltpu.make_async_copy(y_tok.at[pl.ds(0, BBLK)], yt[p], st[p]).wait()

        def start_out(t, p):
            for dh in range(8):
                row = (t * 8 + dh) * NBH + w
                pltpu.async_copy(acc[p].at[pl.ds(dh * 1024, 1024)],
                                 out.at[row], so[p])

        def wait_out(p):
            for _ in range(8):
                pltpu.make_async_copy(acc[p].at[pl.ds(0, 1024)],
                                      out.at[0], so[p]).wait()

        def body(t, p, first):
            if not first:
                wait_tok(p)
            xtoks = [xt[p][pl.ds(l * LANES, LANES)] for l in range(8)]
            ytoks = [yt[p][pl.ds(l * LANES, LANES)] for l in range(8)]
            start_tok(jnp.minimum(t + 2, T - 1), p)
            if not first:
                wait_out(p)
            tspl = jnp.full((LANES,), t, i32)
            ap = acc[p]

            def dstep(d, c):
                dspl = jnp.full((LANES,), d, i32)
                ps = plsc.load_gather(pv, [tspl, dspl])
                gx = [plsc.load_gather(xv, [xtoks[l], dspl]) for l in range(8)]
                gy = [plsc.load_gather(yv, [ytoks[l], dspl]) for l in range(8)]
                ss = [(gx[l] + gy[l]) + ps for l in range(8)]
                for l in range(8):
                    ap[pl.ds(d * BBLK + l * LANES, LANES)] = ss[l]
                return c

            lax.fori_loop(0, EMBED_DIM, dstep, 0)
            start_out(t, p)

        pltpu.sync_copy(x_tok.at[pl.ds(tok_off(0), BBLK)], xt_a)
        pltpu.sync_copy(y_tok.at[pl.ds(tok_off(0), BBLK)], yt_a)
        pltpu.sync_copy(x_tok.at[pl.ds(tok_off(1), BBLK)], xt_b)
        pltpu.sync_copy(y_tok.at[pl.ds(tok_off(1), BBLK)], yt_b)
        body(jnp.int32(0), 0, True)
        body(jnp.int32(1), 1, True)

        def pair(tp, carry):
            body(2 * tp, 0, False)
            body(2 * tp + 1, 1, False)
            return carry

        lax.fori_loop(1, T // 2, pair, 0)

        wait_out(0)
        wait_out(1)
        wait_tok(0)
        wait_tok(1)

    return k


_sc_kernel = _make_kernel()


def kernel(x_tokens, y_tokens, x_table, y_table, pos_table):
    def phys(tok):
        return (tok.astype(jnp.int32)
                .reshape(NBH, BBLK, NTT, 8)
                .transpose(2, 0, 3, 1)
                .reshape(N))

    raw = _sc_kernel(phys(x_tokens), phys(y_tokens),
                     x_table, y_table, pos_table)
    return (raw.reshape(T, 8, NBH, 8, BBLK)
            .transpose(2, 4, 0, 1, 3)
            .reshape(B, T, EMBED_DIM))


# speedup vs baseline: 6.8700x; 6.8700x over previous
import functools

import jax
import jax.numpy as jnp
from jax import lax
from jax.experimental import pallas as pl
from jax.experimental.pallas import tpu as pltpu
from jax.experimental.pallas import tpu_sc as plsc

NBINS_X = 1000
MAX_Y_TOKENS = 201
EMBED_DIM = 64
MAX_LEN = 200
B = 4096
T = 200
N = B * T

LANES = 16
BBLK = 128
NBH = B // BBLK
NTT = T // 8
OUT_ROWS = T * (EMBED_DIM // 8) * NBH


def _make_kernel():
    info = plsc.get_sparse_core_info()
    nc, ns = info.num_cores, info.num_subcores
    nw = nc * ns
    assert nw == NBH

    mesh = plsc.VectorSubcoreMesh(core_axis_name="c", subcore_axis_name="s")

    f32 = jnp.float32
    i32 = jnp.int32

    @functools.partial(
        pl.kernel,
        mesh=mesh,
        out_type=jax.ShapeDtypeStruct((OUT_ROWS, 1024), f32),
        compiler_params=pltpu.CompilerParams(use_tc_tiling_on_sc=False,
                                             needs_layout_passes=False),
        scratch_types=[
            pltpu.VMEM((NBINS_X * EMBED_DIM,), f32),
            pltpu.VMEM((MAX_Y_TOKENS * EMBED_DIM,), f32),
            pltpu.VMEM((MAX_LEN * EMBED_DIM,), f32),
            pltpu.VMEM((BBLK,), i32), pltpu.VMEM((BBLK,), i32),
            pltpu.VMEM((BBLK,), i32), pltpu.VMEM((BBLK,), i32),
            pltpu.VMEM((8 * 1024,), f32), pltpu.VMEM((8 * 1024,), f32),
            pltpu.SemaphoreType.DMA, pltpu.SemaphoreType.DMA,
            pltpu.SemaphoreType.DMA, pltpu.SemaphoreType.DMA,
        ],
    )
    def k(x_tok, y_tok, x_table, y_table, pos_table, out,
          xv, yv, pv, xt_a, xt_b, yt_a, yt_b, acc_a, acc_b,
          st_a, st_b, so_a, so_b):
        w = lax.axis_index("s") * nc + lax.axis_index("c")

        xt = (xt_a, xt_b)
        yt = (yt_a, yt_b)
        acc = (acc_a, acc_b)
        st = (st_a, st_b)
        so = (so_a, so_b)

        pltpu.sync_copy(x_table, xv)
        pltpu.sync_copy(y_table, yv)
        pltpu.sync_copy(pos_table, pv)

        def tok_off(t):
            return ((t // 8) * NBH + w) * 1024 + (t % 8) * BBLK

        def start_tok(t, p):
            off = tok_off(t)
            pltpu.async_copy(x_tok.at[pl.ds(off, BBLK)], xt[p], st[p])
            pltpu.async_copy(y_tok.at[pl.ds(off, BBLK)], yt[p], st[p])

        def wait_tok(p):
            pltpu.make_async_copy(x_tok.at[pl.ds(0, BBLK)], xt[p], st[p]).wait()
            pltpu.make_async_copy(y_tok.at[pl.ds(0, BBLK)], yt[p], st[p]).wait()

        def start_out(t, p):
            for dh in range(8):
                row = (t * 8 + dh) * NBH + w
                pltpu.async_copy(acc[p].at[pl.ds(dh * 1024, 1024)],
                                 out.at[row], so[p])

        def wait_out(p):
            for _ in range(8):
                pltpu.make_async_copy(acc[p].at[pl.ds(0, 1024)],
                                      out.at[0], so[p]).wait()

        def body(t, p, first):
            if not first:
                wait_tok(p)
            xtoks = [xt[p][pl.ds(l * LANES, LANES)] for l in range(8)]
            ytoks = [yt[p][pl.ds(l * LANES, LANES)] for l in range(8)]
            start_tok(jnp.minimum(t + 2, T - 1), p)
            if not first:
                wait_out(p)
            tspl = jnp.full((LANES,), t * EMBED_DIM, i32)
            ap = acc[p]

            def dstep(d, c):
                dbx = jnp.full((LANES,), d * NBINS_X, i32)
                dby = jnp.full((LANES,), d * MAX_Y_TOKENS, i32)
                pidx = tspl + jnp.full((LANES,), d, i32)
                ps = plsc.load_gather(pv, [pidx])
                gx = [plsc.load_gather(xv, [xtoks[l] + dbx]) for l in range(8)]
                gy = [plsc.load_gather(yv, [ytoks[l] + dby]) for l in range(8)]
                ss = [(gx[l] + gy[l]) + ps for l in range(8)]
                for l in range(8):
                    ap[pl.ds(d * BBLK + l * LANES, LANES)] = ss[l]
                return c

            lax.fori_loop(0, EMBED_DIM, dstep, 0)
            start_out(t, p)

        pltpu.sync_copy(x_tok.at[pl.ds(tok_off(0), BBLK)], xt_a)
        pltpu.sync_copy(y_tok.at[pl.ds(tok_off(0), BBLK)], yt_a)
        pltpu.sync_copy(x_tok.at[pl.ds(tok_off(1), BBLK)], xt_b)
        pltpu.sync_copy(y_tok.at[pl.ds(tok_off(1), BBLK)], yt_b)
        body(jnp.int32(0), 0, True)
        body(jnp.int32(1), 1, True)

        def pair(tp, carry):
            body(2 * tp, 0, False)
            body(2 * tp + 1, 1, False)
            return carry

        lax.fori_loop(1, T // 2, pair, 0)

        wait_out(0)
        wait_out(1)
        wait_tok(0)
        wait_tok(1)

    return k


_sc_kernel = _make_kernel()


def kernel(x_tokens, y_tokens, x_table, y_table, pos_table):
    def phys(tok):
        return (tok.astype(jnp.int32)
                .reshape(NBH, BBLK, NTT, 8)
                .transpose(2, 0, 3, 1)
                .reshape(N))

    raw = _sc_kernel(phys(x_tokens), phys(y_tokens),
                     x_table.T.reshape(-1), y_table.T.reshape(-1),
                     pos_table.reshape(-1))
    return (raw.reshape(T, 8, NBH, 8, BBLK)
            .transpose(2, 4, 0, 1, 3)
            .reshape(B, T, EMBED_DIM))


# speedup vs baseline: 7.1332x; 1.0383x over previous
import functools

import jax
import jax.numpy as jnp
from jax import lax
from jax.experimental import pallas as pl
from jax.experimental.pallas import tpu as pltpu
from jax.experimental.pallas import tpu_sc as plsc

NBINS_X = 1000
MAX_Y_TOKENS = 201
EMBED_DIM = 64
MAX_LEN = 200
B = 4096
T = 200
N = B * T

LANES = 16
BBLK = 128
NBH = B // BBLK
NTT = T // 8
OUT_ROWS = T * (EMBED_DIM // 8) * NBH


def _make_kernel():
    info = plsc.get_sparse_core_info()
    nc, ns = info.num_cores, info.num_subcores
    nw = nc * ns
    assert nw == NBH

    mesh = plsc.VectorSubcoreMesh(core_axis_name="c", subcore_axis_name="s")

    f32 = jnp.float32
    i32 = jnp.int32

    @functools.partial(
        pl.kernel,
        mesh=mesh,
        out_type=jax.ShapeDtypeStruct((OUT_ROWS, 1024), f32),
        compiler_params=pltpu.CompilerParams(use_tc_tiling_on_sc=False,
                                             needs_layout_passes=False),
        scratch_types=[
            pltpu.VMEM((NBINS_X * EMBED_DIM,), f32),
            pltpu.VMEM((MAX_Y_TOKENS * EMBED_DIM,), f32),
            pltpu.VMEM((MAX_LEN * EMBED_DIM,), f32),
            pltpu.VMEM((BBLK,), i32), pltpu.VMEM((BBLK,), i32),
            pltpu.VMEM((BBLK,), i32), pltpu.VMEM((BBLK,), i32),
            pltpu.VMEM((8 * 1024,), f32), pltpu.VMEM((8 * 1024,), f32),
            pltpu.SemaphoreType.DMA, pltpu.SemaphoreType.DMA,
            pltpu.SemaphoreType.DMA, pltpu.SemaphoreType.DMA,
        ],
    )
    def k(x_tok, y_tok, x_table, y_table, pos_table, out,
          xv, yv, pv, xt_a, xt_b, yt_a, yt_b, acc_a, acc_b,
          st_a, st_b, so_a, so_b):
        w = lax.axis_index("s") * nc + lax.axis_index("c")

        xt = (xt_a, xt_b)
        yt = (yt_a, yt_b)
        acc = (acc_a, acc_b)
        st = (st_a, st_b)
        so = (so_a, so_b)

        pltpu.sync_copy(x_table, xv)
        pltpu.sync_copy(y_table, yv)
        pltpu.sync_copy(pos_table, pv)

        def tok_off(t):
            return ((t // 8) * NBH + w) * 1024 + (t % 8) * BBLK

        def start_tok(t, p):
            off = tok_off(t)
            pltpu.async_copy(x_tok.at[pl.ds(off, BBLK)], xt[p], st[p])
            pltpu.async_copy(y_tok.at[pl.ds(off, BBLK)], yt[p], st[p])

        def wait_tok(p):
            pltpu.make_async_copy(x_tok.at[pl.ds(0, BBLK)], xt[p], st[p]).wait()
            pltpu.make_async_copy(y_tok.at[pl.ds(0, BBLK)], yt[p], st[p]).wait()

        def start_out(t, p):
            for dh in range(8):
                row = (t * 8 + dh) * NBH + w
                pltpu.async_copy(acc[p].at[pl.ds(dh * 1024, 1024)],
                                 out.at[row], so[p])

        def wait_out(p):
            for _ in range(8):
                pltpu.make_async_copy(acc[p].at[pl.ds(0, 1024)],
                                      out.at[0], so[p]).wait()

        def body(t, p, first):
            if not first:
                wait_tok(p)
            xtoks = [xt[p][pl.ds(l * LANES, LANES)] for l in range(8)]
            ytoks = [yt[p][pl.ds(l * LANES, LANES)] for l in range(8)]
            start_tok(jnp.minimum(t + 2, T - 1), p)
            if not first:
                wait_out(p)
            ap = acc[p]
            prow = [pv[pl.ds(t * EMBED_DIM + j * LANES, LANES)]
                    for j in range(4)]

            for dh in range(4):
                def dstep(dl, c, dh=dh):
                    d = dh * LANES + dl
                    dbx = jnp.full((LANES,), d * NBINS_X, i32)
                    dby = jnp.full((LANES,), d * MAX_Y_TOKENS, i32)
                    dspl = jnp.full((LANES,), dl, i32)
                    ps = prow[dh].at[dspl].get(mode="promise_in_bounds")
                    gx = [plsc.load_gather(xv, [xtoks[l] + dbx])
                          for l in range(8)]
                    gy = [plsc.load_gather(yv, [ytoks[l] + dby])
                          for l in range(8)]
                    ss = [(gx[l] + gy[l]) + ps for l in range(8)]
                    for l in range(8):
                        ap[pl.ds(d * BBLK + l * LANES, LANES)] = ss[l]
                    return c

                lax.fori_loop(0, LANES, dstep, 0)
            start_out(t, p)

        pltpu.sync_copy(x_tok.at[pl.ds(tok_off(0), BBLK)], xt_a)
        pltpu.sync_copy(y_tok.at[pl.ds(tok_off(0), BBLK)], yt_a)
        pltpu.sync_copy(x_tok.at[pl.ds(tok_off(1), BBLK)], xt_b)
        pltpu.sync_copy(y_tok.at[pl.ds(tok_off(1), BBLK)], yt_b)
        body(jnp.int32(0), 0, True)
        body(jnp.int32(1), 1, True)

        def pair(tp, carry):
            body(2 * tp, 0, False)
            body(2 * tp + 1, 1, False)
            return carry

        lax.fori_loop(1, T // 2, pair, 0)

        wait_out(0)
        wait_out(1)
        wait_tok(0)
        wait_tok(1)

    return k


_sc_kernel = _make_kernel()


def kernel(x_tokens, y_tokens, x_table, y_table, pos_table):
    def phys(tok):
        return (tok.astype(jnp.int32)
                .reshape(NBH, BBLK, NTT, 8)
                .transpose(2, 0, 3, 1)
                .reshape(N))

    raw = _sc_kernel(phys(x_tokens), phys(y_tokens),
                     x_table.T.reshape(-1), y_table.T.reshape(-1),
                     pos_table.reshape(-1))
    return (raw.reshape(T, 8, NBH, 8, BBLK)
            .transpose(2, 4, 0, 1, 3)
            .reshape(B, T, EMBED_DIM))
